# baseline (speedup 1.0000x reference)
import functools

import jax
import jax.numpy as jnp
from jax import lax
from jax.experimental import pallas as pl
from jax.experimental.pallas import tpu as pltpu
from jax.experimental.pallas import tpu_sc as plsc

_NC, _NS, _L = 2, 16, 16
_NW = _NC * _NS

_MP = 512

_LN2 = 0.6931471805599453


def _sc_gather(flat, ind_p, cat_p, chw, hw):
    n = ind_p.shape[0]
    per = n // _NW
    shift = _MP.bit_length() - 1

    mesh = plsc.VectorSubcoreMesh(core_axis_name="c", subcore_axis_name="s")

    @functools.partial(
        pl.kernel,
        mesh=mesh,
        out_type=jax.ShapeDtypeStruct((n,), jnp.float32),
        scratch_types=[
            pltpu.VMEM((per,), jnp.int32),
            pltpu.VMEM((per,), jnp.int32),
            pltpu.VMEM((per,), jnp.int32),
            pltpu.VMEM((per,), jnp.float32),
            pltpu.SemaphoreType.DMA,
        ],
    )
    def gather_kernel(flat_hbm, ind_hbm, cat_hbm, out_hbm,
                      ind_v, cat_v, idx_v, val_v, sem):
        wid = lax.axis_index("s") * _NC + lax.axis_index("c")
        base = wid * per
        pltpu.sync_copy(ind_hbm.at[pl.ds(base, per)], ind_v)
        pltpu.sync_copy(cat_hbm.at[pl.ds(base, per)], cat_v)
        for k in range(per // _L):
            off = k * _L
            jv = base + off + lax.iota(jnp.int32, _L)
            bv = lax.shift_right_logical(jv, shift)
            iv = ind_v[pl.ds(off, _L)]
            cv = cat_v[pl.ds(off, _L)]
            idx_v[pl.ds(off, _L)] = bv * chw + cv * hw + iv
        pltpu.async_copy(flat_hbm.at[idx_v], val_v, sem).wait()
        pltpu.sync_copy(val_v, out_hbm.at[pl.ds(base, per)])

    return gather_kernel(flat, ind_p, cat_p)


def _neg_body(o_ref, t_ref, out_ref, acc_ref):
    i = pl.program_id(0)

    @pl.when(i == 0)
    def _init():
        acc_ref[0] = 0.0

    o = jnp.clip(o_ref[...], 0.0001, 1.0 - 0.0001)
    s = 1.0 - t_ref[...]
    s2 = s * s
    acc_ref[0] += jnp.sum(jnp.log2(1.0 - o) * (o * o) * (s2 * s2))

    @pl.when(i == pl.num_programs(0) - 1)
    def _finish():
        out_ref[0] = acc_ref[0]


def _pos_body(pk_ref, mk_ref, neg_ref, out_ref):
    p = jnp.clip(pk_ref[...], 0.0001, 1.0 - 0.0001)
    m = mk_ref[...]
    omp = 1.0 - p
    pos = jnp.sum(jnp.log(p) * (omp * omp) * m)
    num_pos = jnp.sum(m)
    neg = neg_ref[0] * _LN2
    out_ref[0] = jnp.where(num_pos == 0.0, -neg, -(pos + neg) / num_pos)


def kernel(outx, target, ind, mask, cat):
    B, C, H, W = outx.shape
    M = ind.shape[1]
    hw = H * W
    chw = C * hw
    pad = _MP - M

    ind_p = jnp.pad(ind, ((0, 0), (0, pad))).reshape(-1)
    cat_p = jnp.pad(cat, ((0, 0), (0, pad))).reshape(-1)
    mask_p = jnp.pad(mask, ((0, 0), (0, pad)))

    peaks = _sc_gather(outx.reshape(-1), ind_p, cat_p, chw, hw)

    rows = B * C * H
    rb = 8192
    grid = (rows // rb,)

    out2d = outx.reshape(rows, W)
    tgt2d = target.reshape(rows, W)

    neg = pl.pallas_call(
        _neg_body,
        grid=grid,
        in_specs=[
            pl.BlockSpec((rb, W), lambda i: (i, 0)),
            pl.BlockSpec((rb, W), lambda i: (i, 0)),
        ],
        out_specs=pl.BlockSpec(memory_space=pltpu.SMEM),
        out_shape=jax.ShapeDtypeStruct((1,), jnp.float32),
        scratch_shapes=[pltpu.SMEM((1,), jnp.float32)],
    )(out2d, tgt2d)

    res = pl.pallas_call(
        _pos_body,
        in_specs=[
            pl.BlockSpec((B, _MP), lambda: (0, 0)),
            pl.BlockSpec((B, _MP), lambda: (0, 0)),
            pl.BlockSpec(memory_space=pltpu.SMEM),
        ],
        out_specs=pl.BlockSpec(memory_space=pltpu.SMEM),
        out_shape=jax.ShapeDtypeStruct((1,), jnp.float32),
    )(peaks.reshape(B, _MP), mask_p, neg)
    return res[0]


# speedup vs baseline: 1.0759x; 1.0759x over previous
import functools

import jax
import jax.numpy as jnp
from jax import lax
from jax.experimental import pallas as pl
from jax.experimental.pallas import tpu as pltpu
from jax.experimental.pallas import tpu_sc as plsc

_NC, _NS, _L = 2, 16, 16
_NW = _NC * _NS

_MP = 512

_LN2 = 0.6931471805599453


def _sc_gather(flat, ind_p, cat_p, chw, hw):
    n = ind_p.shape[0]
    per = n // _NW
    shift = _MP.bit_length() - 1

    mesh = plsc.VectorSubcoreMesh(core_axis_name="c", subcore_axis_name="s")

    @functools.partial(
        pl.kernel,
        mesh=mesh,
        out_type=jax.ShapeDtypeStruct((n,), jnp.float32),
        scratch_types=[
            pltpu.VMEM((per,), jnp.int32),
            pltpu.VMEM((per,), jnp.int32),
            pltpu.VMEM((per,), jnp.int32),
            pltpu.VMEM((per,), jnp.float32),
            pltpu.SemaphoreType.DMA,
        ],
    )
    def gather_kernel(flat_hbm, ind_hbm, cat_hbm, out_hbm,
                      ind_v, cat_v, idx_v, val_v, sem):
        wid = lax.axis_index("s") * _NC + lax.axis_index("c")
        base = wid * per
        pltpu.sync_copy(ind_hbm.at[pl.ds(base, per)], ind_v)
        pltpu.sync_copy(cat_hbm.at[pl.ds(base, per)], cat_v)
        for k in range(per // _L):
            off = k * _L
            jv = base + off + lax.iota(jnp.int32, _L)
            bv = lax.shift_right_logical(jv, shift)
            iv = ind_v[pl.ds(off, _L)]
            cv = cat_v[pl.ds(off, _L)]
            idx_v[pl.ds(off, _L)] = bv * chw + cv * hw + iv
        pltpu.async_copy(flat_hbm.at[idx_v], val_v, sem).wait()
        pltpu.sync_copy(val_v, out_hbm.at[pl.ds(base, per)])

    return gather_kernel(flat, ind_p, cat_p)


_NBUF = 8
_RB = 2048


def _neg_body(o_hbm, t_hbm, out_ref, obuf, tbuf, acc_ref, osem, tsem):
    i = pl.program_id(0)
    nstep = pl.num_programs(0)
    slot = i % _NBUF

    def _start(step, slot_):
        pltpu.make_async_copy(
            o_hbm.at[pl.ds(step * _RB, _RB)], obuf.at[slot_], osem.at[slot_]
        ).start()
        pltpu.make_async_copy(
            t_hbm.at[pl.ds(step * _RB, _RB)], tbuf.at[slot_], tsem.at[slot_]
        ).start()

    @pl.when(i == 0)
    def _prime():
        acc_ref[0] = 0.0
        for k in range(_NBUF):
            _start(k, k)

    pltpu.make_async_copy(
        o_hbm.at[pl.ds(i * _RB, _RB)], obuf.at[slot], osem.at[slot]
    ).wait()
    pltpu.make_async_copy(
        t_hbm.at[pl.ds(i * _RB, _RB)], tbuf.at[slot], tsem.at[slot]
    ).wait()

    o = obuf[slot]
    l = jnp.log2(1.0 - jnp.clip(o, 0.0001, 1.0 - 0.0001))
    s = 1.0 - tbuf[slot]
    os2 = o * (s * s)
    acc_ref[0] += jnp.sum(l * (os2 * os2))

    nxt = i + _NBUF

    @pl.when(nxt < nstep)
    def _prefetch():
        _start(nxt, slot)

    @pl.when(i == nstep - 1)
    def _finish():
        out_ref[0] = acc_ref[0]


def _pos_body(pk_ref, mk_ref, neg_ref, out_ref):
    p = jnp.clip(pk_ref[...], 0.0001, 1.0 - 0.0001)
    m = mk_ref[...]
    omp = 1.0 - p
    pos = jnp.sum(jnp.log(p) * (omp * omp) * m)
    num_pos = jnp.sum(m)
    neg = neg_ref[0] * _LN2
    out_ref[0] = jnp.where(num_pos == 0.0, -neg, -(pos + neg) / num_pos)


def kernel(outx, target, ind, mask, cat):
    B, C, H, W = outx.shape
    M = ind.shape[1]
    hw = H * W
    chw = C * hw
    pad = _MP - M

    ind_p = jnp.pad(ind, ((0, 0), (0, pad))).reshape(-1)
    cat_p = jnp.pad(cat, ((0, 0), (0, pad))).reshape(-1)
    mask_p = jnp.pad(mask, ((0, 0), (0, pad)))

    peaks = _sc_gather(outx.reshape(-1), ind_p, cat_p, chw, hw)

    rows = B * C * H
    grid = (rows // _RB,)

    out2d = outx.reshape(rows, W)
    tgt2d = target.reshape(rows, W)

    neg = pl.pallas_call(
        _neg_body,
        grid=grid,
        in_specs=[
            pl.BlockSpec(memory_space=pl.ANY),
            pl.BlockSpec(memory_space=pl.ANY),
        ],
        out_specs=pl.BlockSpec(memory_space=pltpu.SMEM),
        out_shape=jax.ShapeDtypeStruct((1,), jnp.float32),
        scratch_shapes=[
            pltpu.VMEM((_NBUF, _RB, W), jnp.float32),
            pltpu.VMEM((_NBUF, _RB, W), jnp.float32),
            pltpu.SMEM((1,), jnp.float32),
            pltpu.SemaphoreType.DMA((_NBUF,)),
            pltpu.SemaphoreType.DMA((_NBUF,)),
        ],
    )(out2d, tgt2d)

    res = pl.pallas_call(
        _pos_body,
        in_specs=[
            pl.BlockSpec((B, _MP), lambda: (0, 0)),
            pl.BlockSpec((B, _MP), lambda: (0, 0)),
            pl.BlockSpec(memory_space=pltpu.SMEM),
        ],
        out_specs=pl.BlockSpec(memory_space=pltpu.SMEM),
        out_shape=jax.ShapeDtypeStruct((1,), jnp.float32),
    )(peaks.reshape(B, _MP), mask_p, neg)
    return res[0]


# speedup vs baseline: 1.0910x; 1.0141x over previous
import functools

import jax
import jax.numpy as jnp
from jax import lax
from jax.experimental import pallas as pl
from jax.experimental.pallas import tpu as pltpu
from jax.experimental.pallas import tpu_sc as plsc

_NC, _NS, _L = 2, 16, 16
_NW = _NC * _NS

_LN2 = 0.6931471805599453

_NBUF = 8
_RB = 2048


def _sc_gather(flat, ind_f, cat_f, mask_f, chw, hw, m_per_b):
    n = ind_f.shape[0]
    pad_per = 128
    npad = _NW * pad_per

    mesh = plsc.VectorSubcoreMesh(core_axis_name="c", subcore_axis_name="s")

    @functools.partial(
        pl.kernel,
        mesh=mesh,
        out_type=[
            jax.ShapeDtypeStruct((npad,), jnp.float32),
            jax.ShapeDtypeStruct((npad,), jnp.float32),
        ],
        scratch_types=[
            pltpu.VMEM((pad_per,), jnp.int32),
            pltpu.VMEM((pad_per,), jnp.int32),
            pltpu.VMEM((pad_per,), jnp.float32),
            pltpu.VMEM((pad_per,), jnp.int32),
            pltpu.VMEM((pad_per,), jnp.float32),
            pltpu.VMEM((pad_per,), jnp.float32),
            pltpu.SemaphoreType.DMA,
        ],
    )
    def gather_kernel(flat_hbm, ind_hbm, cat_hbm, mask_hbm,
                      pk_hbm, mk_hbm,
                      ind_v, cat_v, mask_v, idx_v, mko_v, val_v, sem):
        wid = lax.axis_index("s") * _NC + lax.axis_index("c")
        own = wid * pad_per
        start = pl.multiple_of(jnp.minimum(own, n - pad_per), 8)
        pltpu.sync_copy(ind_hbm.at[pl.ds(start, pad_per)], ind_v)
        pltpu.sync_copy(cat_hbm.at[pl.ds(start, pad_per)], cat_v)
        pltpu.sync_copy(mask_hbm.at[pl.ds(start, pad_per)], mask_v)
        for k in range(pad_per // _L):
            off = k * _L
            j = off + lax.iota(jnp.int32, _L)
            g = start + j
            valid = g >= own
            iv = ind_v[pl.ds(off, _L)]
            cv = cat_v[pl.ds(off, _L)]
            mv = mask_v[pl.ds(off, _L)]
            bv = jnp.zeros((_L,), jnp.int32)
            for kk in range(1, n // m_per_b):
                bv = bv + jnp.where(g >= kk * m_per_b, 1, 0)
            idx_v[pl.ds(off, _L)] = jnp.where(
                valid, bv * chw + cv * hw + iv, 0)
            mko_v[pl.ds(off, _L)] = jnp.where(valid, mv, 0.0)
        pltpu.async_copy(flat_hbm.at[idx_v], val_v, sem).wait()
        pltpu.sync_copy(val_v, pk_hbm.at[pl.ds(own, pad_per)])
        pltpu.sync_copy(mko_v, mk_hbm.at[pl.ds(own, pad_per)])

    return gather_kernel(flat, ind_f, cat_f, mask_f)


def _neg_body(o_hbm, t_hbm, out_ref, obuf, tbuf, acc_ref, osem, tsem):
    i = pl.program_id(0)
    nstep = pl.num_programs(0)
    slot = i % _NBUF

    def _start(step, slot_):
        pltpu.make_async_copy(
            o_hbm.at[pl.ds(step * _RB, _RB)], obuf.at[slot_], osem.at[slot_]
        ).start()
        pltpu.make_async_copy(
            t_hbm.at[pl.ds(step * _RB, _RB)], tbuf.at[slot_], tsem.at[slot_]
        ).start()

    @pl.when(i == 0)
    def _prime():
        acc_ref[0] = 0.0
        for k in range(_NBUF):
            _start(k, k)

    pltpu.make_async_copy(
        o_hbm.at[pl.ds(i * _RB, _RB)], obuf.at[slot], osem.at[slot]
    ).wait()
    pltpu.make_async_copy(
        t_hbm.at[pl.ds(i * _RB, _RB)], tbuf.at[slot], tsem.at[slot]
    ).wait()

    o = obuf[slot]
    l = jnp.log2(1.0 - jnp.clip(o, 0.0001, 1.0 - 0.0001))
    s = 1.0 - tbuf[slot]
    os2 = o * (s * s)
    acc_ref[0] += jnp.sum(l * (os2 * os2))

    nxt = i + _NBUF

    @pl.when(nxt < nstep)
    def _prefetch():
        _start(nxt, slot)

    @pl.when(i == nstep - 1)
    def _finish():
        out_ref[0] = acc_ref[0]


def _pos_body(pk_ref, mk_ref, neg_ref, out_ref):
    p = jnp.clip(pk_ref[...], 0.0001, 1.0 - 0.0001)
    m = mk_ref[...]
    omp = 1.0 - p
    pos = jnp.sum(jnp.log(p) * (omp * omp) * m)
    num_pos = jnp.sum(m)
    neg = neg_ref[0] * _LN2
    out_ref[0] = jnp.where(num_pos == 0.0, -neg, -(pos + neg) / num_pos)


def kernel(outx, target, ind, mask, cat):
    B, C, H, W = outx.shape
    M = ind.shape[1]
    hw = H * W
    chw = C * hw

    peaks, maskp = _sc_gather(
        outx.reshape(-1), ind.reshape(-1), cat.reshape(-1),
        mask.reshape(-1), chw, hw, M)

    rows = B * C * H
    grid = (rows // _RB,)

    out2d = outx.reshape(rows, W)
    tgt2d = target.reshape(rows, W)

    neg = pl.pallas_call(
        _neg_body,
        grid=grid,
        in_specs=[
            pl.BlockSpec(memory_space=pl.ANY),
            pl.BlockSpec(memory_space=pl.ANY),
        ],
        out_specs=pl.BlockSpec(memory_space=pltpu.SMEM),
        out_shape=jax.ShapeDtypeStruct((1,), jnp.float32),
        scratch_shapes=[
            pltpu.VMEM((_NBUF, _RB, W), jnp.float32),
            pltpu.VMEM((_NBUF, _RB, W), jnp.float32),
            pltpu.SMEM((1,), jnp.float32),
            pltpu.SemaphoreType.DMA((_NBUF,)),
            pltpu.SemaphoreType.DMA((_NBUF,)),
        ],
    )(out2d, tgt2d)

    npk = peaks.shape[0]
    res = pl.pallas_call(
        _pos_body,
        in_specs=[
            pl.BlockSpec((npk // 128, 128), lambda: (0, 0)),
            pl.BlockSpec((npk // 128, 128), lambda: (0, 0)),
            pl.BlockSpec(memory_space=pltpu.SMEM),
        ],
        out_specs=pl.BlockSpec(memory_space=pltpu.SMEM),
        out_shape=jax.ShapeDtypeStruct((1,), jnp.float32),
    )(peaks.reshape(npk // 128, 128), maskp.reshape(npk // 128, 128), neg)
    return res[0]


# speedup vs baseline: 1.1439x; 1.0484x over previous
import functools

import jax
import jax.numpy as jnp
from jax import lax
from jax.experimental import pallas as pl
from jax.experimental.pallas import tpu as pltpu
from jax.experimental.pallas import tpu_sc as plsc

_NC, _NS, _L = 2, 16, 16
_NW = _NC * _NS

_LN2 = 0.6931471805599453

_NBUF = 8
_RB = 2048


def _sc_gather(flat, icm, chw, hw, m_per_b):
    n = icm.shape[0] // 3
    pad_per = 128
    npad = _NW * pad_per

    mesh = plsc.VectorSubcoreMesh(core_axis_name="c", subcore_axis_name="s")

    @functools.partial(
        pl.kernel,
        mesh=mesh,
        out_type=[
            jax.ShapeDtypeStruct((npad,), jnp.float32),
            jax.ShapeDtypeStruct((npad,), jnp.int32),
        ],
        scratch_types=[
            pltpu.VMEM((pad_per,), jnp.int32),
            pltpu.VMEM((pad_per,), jnp.int32),
            pltpu.VMEM((pad_per,), jnp.int32),
            pltpu.VMEM((pad_per,), jnp.int32),
            pltpu.VMEM((pad_per,), jnp.int32),
            pltpu.VMEM((pad_per,), jnp.float32),
            pltpu.SemaphoreType.DMA,
        ],
    )
    def gather_kernel(flat_hbm, icm_hbm,
                      pk_hbm, mk_hbm,
                      ind_v, cat_v, mask_v, idx_v, mko_v, val_v, sem):
        wid = lax.axis_index("s") * _NC + lax.axis_index("c")
        own = wid * pad_per
        start = pl.multiple_of(jnp.minimum(own, n - pad_per), 8)
        pltpu.sync_copy(icm_hbm.at[pl.ds(start, pad_per)], ind_v)
        pltpu.sync_copy(icm_hbm.at[pl.ds(n + start, pad_per)], cat_v)
        pltpu.sync_copy(icm_hbm.at[pl.ds(2 * n + start, pad_per)], mask_v)
        for k in range(pad_per // _L):
            off = k * _L
            j = off + lax.iota(jnp.int32, _L)
            g = start + j
            valid = g >= own
            iv = ind_v[pl.ds(off, _L)]
            cv = cat_v[pl.ds(off, _L)]
            mv = mask_v[pl.ds(off, _L)]
            bv = jnp.zeros((_L,), jnp.int32)
            for kk in range(1, n // m_per_b):
                bv = bv + jnp.where(g >= kk * m_per_b, 1, 0)
            idx_v[pl.ds(off, _L)] = jnp.where(
                valid, bv * chw + cv * hw + iv, 0)
            mko_v[pl.ds(off, _L)] = jnp.where(valid, mv, 0)
        pltpu.async_copy(flat_hbm.at[idx_v], val_v, sem).wait()
        pltpu.sync_copy(val_v, pk_hbm.at[pl.ds(own, pad_per)])
        pltpu.sync_copy(mko_v, mk_hbm.at[pl.ds(own, pad_per)])

    return gather_kernel(flat, icm)


def _neg_body(o_hbm, t_hbm, out_ref, obuf, tbuf, acc_ref, osem, tsem):
    i = pl.program_id(0)
    nstep = pl.num_programs(0)
    slot = i % _NBUF

    def _start(step, slot_):
        pltpu.make_async_copy(
            o_hbm.at[pl.ds(step * _RB, _RB)], obuf.at[slot_], osem.at[slot_]
        ).start()
        pltpu.make_async_copy(
            t_hbm.at[pl.ds(step * _RB, _RB)], tbuf.at[slot_], tsem.at[slot_]
        ).start()

    @pl.when(i == 0)
    def _prime():
        acc_ref[0] = 0.0
        for k in range(_NBUF):
            _start(k, k)

    pltpu.make_async_copy(
        o_hbm.at[pl.ds(i * _RB, _RB)], obuf.at[slot], osem.at[slot]
    ).wait()
    pltpu.make_async_copy(
        t_hbm.at[pl.ds(i * _RB, _RB)], tbuf.at[slot], tsem.at[slot]
    ).wait()

    o = obuf[slot]
    l = jnp.log2(jnp.maximum(1.0 - o, 0.0001))
    s = 1.0 - tbuf[slot]
    os2 = o * (s * s)
    acc_ref[0] += jnp.sum(l * (os2 * os2))

    nxt = i + _NBUF

    @pl.when(nxt < nstep)
    def _prefetch():
        _start(nxt, slot)

    @pl.when(i == nstep - 1)
    def _finish():
        out_ref[0] = acc_ref[0]


def _pos_body(pk_ref, mk_ref, neg_ref, out_ref):
    p = jnp.clip(pk_ref[...], 0.0001, 1.0 - 0.0001)
    m = lax.bitcast_convert_type(mk_ref[...], jnp.float32)
    omp = 1.0 - p
    pos = jnp.sum(jnp.log(p) * (omp * omp) * m)
    num_pos = jnp.sum(m)
    neg = neg_ref[0] * _LN2
    out_ref[0] = jnp.where(num_pos == 0.0, -neg, -(pos + neg) / num_pos)


def kernel(outx, target, ind, mask, cat):
    B, C, H, W = outx.shape
    M = ind.shape[1]
    hw = H * W
    chw = C * hw

    icm = jnp.concatenate([
        ind.reshape(-1), cat.reshape(-1),
        lax.bitcast_convert_type(mask, jnp.int32).reshape(-1)])
    peaks, maskp = _sc_gather(outx.reshape(-1), icm, chw, hw, M)

    rows = B * C * H
    grid = (rows // _RB,)

    out2d = outx.reshape(rows, W)
    tgt2d = target.reshape(rows, W)

    neg = pl.pallas_call(
        _neg_body,
        grid=grid,
        in_specs=[
            pl.BlockSpec(memory_space=pl.ANY),
            pl.BlockSpec(memory_space=pl.ANY),
        ],
        out_specs=pl.BlockSpec(memory_space=pltpu.SMEM),
        out_shape=jax.ShapeDtypeStruct((1,), jnp.float32),
        scratch_shapes=[
            pltpu.VMEM((_NBUF, _RB, W), jnp.float32),
            pltpu.VMEM((_NBUF, _RB, W), jnp.float32),
            pltpu.SMEM((1,), jnp.float32),
            pltpu.SemaphoreType.DMA((_NBUF,)),
            pltpu.SemaphoreType.DMA((_NBUF,)),
        ],
    )(out2d, tgt2d)

    npk = peaks.shape[0]
    res = pl.pallas_call(
        _pos_body,
        in_specs=[
            pl.BlockSpec((npk // 128, 128), lambda: (0, 0)),
            pl.BlockSpec((npk // 128, 128), lambda: (0, 0)),
            pl.BlockSpec(memory_space=pltpu.SMEM),
        ],
        out_specs=pl.BlockSpec(memory_space=pltpu.SMEM),
        out_shape=jax.ShapeDtypeStruct((1,), jnp.float32),
    )(peaks.reshape(npk // 128, 128), maskp.reshape(npk // 128, 128), neg)
    return res[0]


# speedup vs baseline: 1.2063x; 1.0545x over previous
import functools

import jax
import jax.numpy as jnp
from jax import lax
from jax.experimental import pallas as pl
from jax.experimental.pallas import tpu as pltpu
from jax.experimental.pallas import tpu_sc as plsc

_NC, _NS, _L = 2, 16, 16
_NW = _NC * _NS

_LN2 = 0.6931471805599453

_NBUF = 6
_RB = 4096


def _sc_gather(flat, icm, chw, hw, m_per_b):
    n = icm.shape[0] // 3
    ncores = 1
    nw = ncores * _NS
    npad = 4096
    pad_per = npad // nw

    mesh = plsc.VectorSubcoreMesh(
        core_axis_name="c", subcore_axis_name="s", num_cores=ncores)

    @functools.partial(
        pl.kernel,
        mesh=mesh,
        out_type=[
            jax.ShapeDtypeStruct((npad,), jnp.float32),
            jax.ShapeDtypeStruct((npad,), jnp.int32),
        ],
        scratch_types=[
            pltpu.VMEM((pad_per,), jnp.int32),
            pltpu.VMEM((pad_per,), jnp.int32),
            pltpu.VMEM((pad_per,), jnp.int32),
            pltpu.VMEM((pad_per,), jnp.int32),
            pltpu.VMEM((pad_per,), jnp.int32),
            pltpu.VMEM((pad_per,), jnp.float32),
            pltpu.SemaphoreType.DMA,
        ],
    )
    def gather_kernel(flat_hbm, icm_hbm,
                      pk_hbm, mk_hbm,
                      ind_v, cat_v, mask_v, idx_v, mko_v, val_v, sem):
        wid = lax.axis_index("s") * ncores + lax.axis_index("c")
        own = wid * pad_per
        start = pl.multiple_of(jnp.minimum(own, n - pad_per), 8)
        pltpu.sync_copy(icm_hbm.at[pl.ds(start, pad_per)], ind_v)
        pltpu.sync_copy(icm_hbm.at[pl.ds(n + start, pad_per)], cat_v)
        pltpu.sync_copy(icm_hbm.at[pl.ds(2 * n + start, pad_per)], mask_v)
        for k in range(pad_per // _L):
            off = k * _L
            j = off + lax.iota(jnp.int32, _L)
            g = start + j
            valid = g >= own
            iv = ind_v[pl.ds(off, _L)]
            cv = cat_v[pl.ds(off, _L)]
            mv = mask_v[pl.ds(off, _L)]
            bv = jnp.zeros((_L,), jnp.int32)
            for kk in range(1, n // m_per_b):
                bv = bv + jnp.where(g >= kk * m_per_b, 1, 0)
            idx_v[pl.ds(off, _L)] = jnp.where(
                valid, bv * chw + cv * hw + iv, 0)
            mko_v[pl.ds(off, _L)] = jnp.where(valid, mv, 0)
        pltpu.async_copy(flat_hbm.at[idx_v], val_v, sem).wait()
        pltpu.sync_copy(val_v, pk_hbm.at[pl.ds(own, pad_per)])
        pltpu.sync_copy(mko_v, mk_hbm.at[pl.ds(own, pad_per)])

    return gather_kernel(flat, icm)


def _neg_body(o_hbm, t_hbm, out_ref, obuf, tbuf, acc_ref, osem, tsem):
    i = pl.program_id(0)
    nstep = pl.num_programs(0)
    slot = i % _NBUF

    def _start(step, slot_):
        pltpu.make_async_copy(
            o_hbm.at[pl.ds(step * _RB, _RB)], obuf.at[slot_], osem.at[slot_]
        ).start()
        pltpu.make_async_copy(
            t_hbm.at[pl.ds(step * _RB, _RB)], tbuf.at[slot_], tsem.at[slot_]
        ).start()

    @pl.when(i == 0)
    def _prime():
        acc_ref[0] = 0.0
        for k in range(_NBUF):
            _start(k, k)

    pltpu.make_async_copy(
        o_hbm.at[pl.ds(i * _RB, _RB)], obuf.at[slot], osem.at[slot]
    ).wait()
    pltpu.make_async_copy(
        t_hbm.at[pl.ds(i * _RB, _RB)], tbuf.at[slot], tsem.at[slot]
    ).wait()

    o = obuf[slot]
    l = jnp.log2(jnp.maximum(1.0 - o, 0.0001))
    s = 1.0 - tbuf[slot]
    os2 = o * (s * s)
    acc_ref[0] += jnp.sum(l * (os2 * os2))

    nxt = i + _NBUF

    @pl.when(nxt < nstep)
    def _prefetch():
        _start(nxt, slot)

    @pl.when(i == nstep - 1)
    def _finish():
        out_ref[0] = acc_ref[0]


def _pos_body(pk_ref, mk_ref, neg_ref, out_ref):
    p = jnp.clip(pk_ref[...], 0.0001, 1.0 - 0.0001)
    m = lax.bitcast_convert_type(mk_ref[...], jnp.float32)
    omp = 1.0 - p
    pos = jnp.sum(jnp.log(p) * (omp * omp) * m)
    num_pos = jnp.sum(m)
    neg = neg_ref[0] * _LN2
    out_ref[0] = jnp.where(num_pos == 0.0, -neg, -(pos + neg) / num_pos)


def kernel(outx, target, ind, mask, cat):
    B, C, H, W = outx.shape
    M = ind.shape[1]
    hw = H * W
    chw = C * hw

    icm = jnp.concatenate([
        ind.reshape(-1), cat.reshape(-1),
        lax.bitcast_convert_type(mask, jnp.int32).reshape(-1)])
    peaks, maskp = _sc_gather(outx.reshape(-1), icm, chw, hw, M)

    rows = B * C * H
    grid = (rows // _RB,)

    out2d = outx.reshape(rows, W)
    tgt2d = target.reshape(rows, W)

    neg = pl.pallas_call(
        _neg_body,
        grid=grid,
        in_specs=[
            pl.BlockSpec(memory_space=pl.ANY),
            pl.BlockSpec(memory_space=pl.ANY),
        ],
        out_specs=pl.BlockSpec(memory_space=pltpu.SMEM),
        out_shape=jax.ShapeDtypeStruct((1,), jnp.float32),
        scratch_shapes=[
            pltpu.VMEM((_NBUF, _RB, W), jnp.float32),
            pltpu.VMEM((_NBUF, _RB, W), jnp.float32),
            pltpu.SMEM((1,), jnp.float32),
            pltpu.SemaphoreType.DMA((_NBUF,)),
            pltpu.SemaphoreType.DMA((_NBUF,)),
        ],
    )(out2d, tgt2d)

    npk = peaks.shape[0]
    res = pl.pallas_call(
        _pos_body,
        in_specs=[
            pl.BlockSpec((npk // 128, 128), lambda: (0, 0)),
            pl.BlockSpec((npk // 128, 128), lambda: (0, 0)),
            pl.BlockSpec(memory_space=pltpu.SMEM),
        ],
        out_specs=pl.BlockSpec(memory_space=pltpu.SMEM),
        out_shape=jax.ShapeDtypeStruct((1,), jnp.float32),
    )(peaks.reshape(npk // 128, 128), maskp.reshape(npk // 128, 128), neg)
    return res[0]


# speedup vs baseline: 1.2251x; 1.0156x over previous
import functools

import jax
import jax.numpy as jnp
from jax import lax
from jax.experimental import pallas as pl
from jax.experimental.pallas import tpu as pltpu
from jax.experimental.pallas import tpu_sc as plsc

_NC, _NS, _L = 2, 16, 16
_NW = _NC * _NS

_LN2 = 0.6931471805599453

_NBUF = 7
_RB = 4096


def _sc_gather(flat, icm, chw, hw, m_per_b):
    n = icm.shape[0] // 3
    ncores = 1
    nw = ncores * _NS
    npad = 4096
    pad_per = npad // nw

    mesh = plsc.VectorSubcoreMesh(
        core_axis_name="c", subcore_axis_name="s", num_cores=ncores)

    @functools.partial(
        pl.kernel,
        mesh=mesh,
        out_type=[
            jax.ShapeDtypeStruct((npad,), jnp.float32),
            jax.ShapeDtypeStruct((npad,), jnp.int32),
        ],
        scratch_types=[
            pltpu.VMEM((pad_per,), jnp.int32),
            pltpu.VMEM((pad_per,), jnp.int32),
            pltpu.VMEM((pad_per,), jnp.int32),
            pltpu.VMEM((pad_per,), jnp.int32),
            pltpu.VMEM((pad_per,), jnp.int32),
            pltpu.VMEM((pad_per,), jnp.float32),
            pltpu.SemaphoreType.DMA,
        ],
    )
    def gather_kernel(flat_hbm, icm_hbm,
                      pk_hbm, mk_hbm,
                      ind_v, cat_v, mask_v, idx_v, mko_v, val_v, sem):
        wid = lax.axis_index("s") * ncores + lax.axis_index("c")
        own = wid * pad_per
        start = pl.multiple_of(jnp.minimum(own, n - pad_per), 8)
        pltpu.sync_copy(icm_hbm.at[pl.ds(start, pad_per)], ind_v)
        pltpu.sync_copy(icm_hbm.at[pl.ds(n + start, pad_per)], cat_v)
        pltpu.sync_copy(icm_hbm.at[pl.ds(2 * n + start, pad_per)], mask_v)
        for k in range(pad_per // _L):
            off = k * _L
            j = off + lax.iota(jnp.int32, _L)
            g = start + j
            valid = g >= own
            iv = ind_v[pl.ds(off, _L)]
            cv = cat_v[pl.ds(off, _L)]
            mv = mask_v[pl.ds(off, _L)]
            bv = jnp.zeros((_L,), jnp.int32)
            for kk in range(1, n // m_per_b):
                bv = bv + jnp.where(g >= kk * m_per_b, 1, 0)
            idx_v[pl.ds(off, _L)] = jnp.where(
                valid, bv * chw + cv * hw + iv, 0)
            mko_v[pl.ds(off, _L)] = jnp.where(valid, mv, 0)
        pltpu.async_copy(flat_hbm.at[idx_v], val_v, sem).wait()
        pltpu.sync_copy(val_v, pk_hbm.at[pl.ds(own, pad_per)])
        pltpu.sync_copy(mko_v, mk_hbm.at[pl.ds(own, pad_per)])

    return gather_kernel(flat, icm)


def _neg_body(o_hbm, t_hbm, out_ref, obuf, tbuf, acc_ref, osem, tsem):
    i = pl.program_id(0)
    nstep = pl.num_programs(0)
    slot = i % _NBUF

    def _start(step, slot_):
        pltpu.make_async_copy(
            o_hbm.at[pl.ds(step * _RB, _RB)], obuf.at[slot_], osem.at[slot_]
        ).start()
        pltpu.make_async_copy(
            t_hbm.at[pl.ds(step * _RB, _RB)], tbuf.at[slot_], tsem.at[slot_]
        ).start()

    @pl.when(i == 0)
    def _prime():
        acc_ref[0] = 0.0
        for k in range(_NBUF):
            _start(k, k)

    pltpu.make_async_copy(
        o_hbm.at[pl.ds(i * _RB, _RB)], obuf.at[slot], osem.at[slot]
    ).wait()
    pltpu.make_async_copy(
        t_hbm.at[pl.ds(i * _RB, _RB)], tbuf.at[slot], tsem.at[slot]
    ).wait()

    o = obuf[slot]
    l = jnp.log2(jnp.maximum(1.0 - o, 0.0001))
    s = 1.0 - tbuf[slot]
    os2 = o * (s * s)
    acc_ref[0] += jnp.sum(l * (os2 * os2))

    nxt = i + _NBUF

    @pl.when(nxt < nstep)
    def _prefetch():
        _start(nxt, slot)

    @pl.when(i == nstep - 1)
    def _finish():
        out_ref[0] = acc_ref[0]


def _pos_body(pk_ref, mk_ref, neg_ref, out_ref):
    p = jnp.clip(pk_ref[...], 0.0001, 1.0 - 0.0001)
    m = lax.bitcast_convert_type(mk_ref[...], jnp.float32)
    omp = 1.0 - p
    pos = jnp.sum(jnp.log(p) * (omp * omp) * m)
    num_pos = jnp.sum(m)
    neg = neg_ref[0] * _LN2
    out_ref[0] = jnp.where(num_pos == 0.0, -neg, -(pos + neg) / num_pos)


def kernel(outx, target, ind, mask, cat):
    B, C, H, W = outx.shape
    M = ind.shape[1]
    hw = H * W
    chw = C * hw

    icm = jnp.concatenate(
        [ind, cat, lax.bitcast_convert_type(mask, jnp.int32)],
        axis=0).reshape(-1)
    peaks, maskp = _sc_gather(outx.reshape(-1), icm, chw, hw, M)

    rows = B * C * H
    grid = (rows // _RB,)

    out2d = outx.reshape(rows, W)
    tgt2d = target.reshape(rows, W)

    neg = pl.pallas_call(
        _neg_body,
        grid=grid,
        in_specs=[
            pl.BlockSpec(memory_space=pl.ANY),
            pl.BlockSpec(memory_space=pl.ANY),
        ],
        out_specs=pl.BlockSpec(memory_space=pltpu.SMEM),
        out_shape=jax.ShapeDtypeStruct((1,), jnp.float32),
        scratch_shapes=[
            pltpu.VMEM((_NBUF, _RB, W), jnp.float32),
            pltpu.VMEM((_NBUF, _RB, W), jnp.float32),
            pltpu.SMEM((1,), jnp.float32),
            pltpu.SemaphoreType.DMA((_NBUF,)),
            pltpu.SemaphoreType.DMA((_NBUF,)),
        ],
    )(out2d, tgt2d)

    npk = peaks.shape[0]
    res = pl.pallas_call(
        _pos_body,
        in_specs=[
            pl.BlockSpec((npk // 128, 128), lambda: (0, 0)),
            pl.BlockSpec((npk // 128, 128), lambda: (0, 0)),
            pl.BlockSpec(memory_space=pltpu.SMEM),
        ],
        out_specs=pl.BlockSpec(memory_space=pltpu.SMEM),
        out_shape=jax.ShapeDtypeStruct((1,), jnp.float32),
    )(peaks.reshape(npk // 128, 128), maskp.reshape(npk // 128, 128), neg)
    return res[0]


# speedup vs baseline: 1.2261x; 1.0008x over previous
import functools

import jax
import jax.numpy as jnp
from jax import lax
from jax.experimental import pallas as pl
from jax.experimental.pallas import tpu as pltpu
from jax.experimental.pallas import tpu_sc as plsc

_NS, _L = 16, 16

_LN2 = 0.6931471805599453

_NBUF = 7
_RB = 4096


def _sc_gather(flat, icm, chw, hw, m_per_b):
    n = icm.shape[0] // 3
    ncores = 1
    nw = ncores * _NS
    npad = 4096
    pad_per = npad // nw

    mesh = plsc.VectorSubcoreMesh(
        core_axis_name="c", subcore_axis_name="s", num_cores=ncores)

    @functools.partial(
        pl.kernel,
        mesh=mesh,
        out_type=[
            jax.ShapeDtypeStruct((npad,), jnp.float32),
            jax.ShapeDtypeStruct((npad,), jnp.int32),
        ],
        scratch_types=[
            pltpu.VMEM((pad_per,), jnp.int32),
            pltpu.VMEM((pad_per,), jnp.int32),
            pltpu.VMEM((pad_per,), jnp.int32),
            pltpu.VMEM((pad_per,), jnp.int32),
            pltpu.VMEM((pad_per,), jnp.int32),
            pltpu.VMEM((pad_per,), jnp.float32),
            pltpu.SemaphoreType.DMA,
        ],
    )
    def gather_kernel(flat_hbm, icm_hbm,
                      pk_hbm, mk_hbm,
                      ind_v, cat_v, mask_v, idx_v, mko_v, val_v, sem):
        wid = lax.axis_index("s") * ncores + lax.axis_index("c")
        own = wid * pad_per
        start = pl.multiple_of(jnp.minimum(own, n - pad_per), 8)
        pltpu.sync_copy(icm_hbm.at[pl.ds(start, pad_per)], ind_v)
        pltpu.sync_copy(icm_hbm.at[pl.ds(n + start, pad_per)], cat_v)
        pltpu.sync_copy(icm_hbm.at[pl.ds(2 * n + start, pad_per)], mask_v)
        for k in range(pad_per // _L):
            off = k * _L
            j = off + lax.iota(jnp.int32, _L)
            g = start + j
            valid = g >= own
            iv = ind_v[pl.ds(off, _L)]
            cv = cat_v[pl.ds(off, _L)]
            mv = mask_v[pl.ds(off, _L)]
            bv = jnp.zeros((_L,), jnp.int32)
            for kk in range(1, n // m_per_b):
                bv = bv + jnp.where(g >= kk * m_per_b, 1, 0)
            idx_v[pl.ds(off, _L)] = jnp.where(
                valid, bv * chw + cv * hw + iv, 0)
            mko_v[pl.ds(off, _L)] = jnp.where(valid, mv, 0)
        pltpu.async_copy(flat_hbm.at[idx_v], val_v, sem).wait()
        pltpu.sync_copy(val_v, pk_hbm.at[pl.ds(own, pad_per)])
        pltpu.sync_copy(mko_v, mk_hbm.at[pl.ds(own, pad_per)])

    return gather_kernel(flat, icm)


def _neg_body(o_hbm, t_hbm, out_ref, obuf, tbuf, acc_ref, osem, tsem):
    i = pl.program_id(0)
    nstep = pl.num_programs(0)
    slot = i % _NBUF

    def _start(step, slot_):
        pltpu.make_async_copy(
            o_hbm.at[pl.ds(step * _RB, _RB)], obuf.at[slot_], osem.at[slot_]
        ).start()
        pltpu.make_async_copy(
            t_hbm.at[pl.ds(step * _RB, _RB)], tbuf.at[slot_], tsem.at[slot_]
        ).start()

    @pl.when(i == 0)
    def _prime():
        acc_ref[0] = 0.0
        for k in range(_NBUF):
            _start(k, k)

    pltpu.make_async_copy(
        o_hbm.at[pl.ds(i * _RB, _RB)], obuf.at[slot], osem.at[slot]
    ).wait()
    pltpu.make_async_copy(
        t_hbm.at[pl.ds(i * _RB, _RB)], tbuf.at[slot], tsem.at[slot]
    ).wait()

    o = obuf[slot]
    l = jnp.log2(jnp.maximum(1.0 - o, 0.0001))
    s = 1.0 - tbuf[slot]
    os2 = o * (s * s)
    acc_ref[0] += jnp.sum(l * (os2 * os2))

    nxt = i + _NBUF

    @pl.when(nxt < nstep)
    def _prefetch():
        _start(nxt, slot)

    @pl.when(i == nstep - 1)
    def _finish():
        out_ref[0] = acc_ref[0]


def _pos_body(pk_ref, mk_ref, neg_ref, out_ref):
    p = jnp.clip(pk_ref[...], 0.0001, 1.0 - 0.0001)
    m = lax.bitcast_convert_type(mk_ref[...], jnp.float32)
    omp = 1.0 - p
    pos = jnp.sum(jnp.log(p) * (omp * omp) * m)
    num_pos = jnp.sum(m)
    neg = neg_ref[0] * _LN2
    out_ref[0] = jnp.where(num_pos == 0.0, -neg, -(pos + neg) / num_pos)


def kernel(outx, target, ind, mask, cat):
    B, C, H, W = outx.shape
    M = ind.shape[1]
    hw = H * W
    chw = C * hw

    icm = jnp.concatenate(
        [ind, cat, lax.bitcast_convert_type(mask, jnp.int32)],
        axis=0).reshape(-1)
    peaks, maskp = _sc_gather(outx.reshape(-1), icm, chw, hw, M)

    rows = B * C * H
    grid = (rows // _RB,)

    out2d = outx.reshape(rows, W)
    tgt2d = target.reshape(rows, W)

    neg = pl.pallas_call(
        _neg_body,
        grid=grid,
        in_specs=[
            pl.BlockSpec(memory_space=pl.ANY),
            pl.BlockSpec(memory_space=pl.ANY),
        ],
        out_specs=pl.BlockSpec(memory_space=pltpu.SMEM),
        out_shape=jax.ShapeDtypeStruct((1,), jnp.float32),
        scratch_shapes=[
            pltpu.VMEM((_NBUF, _RB, W), jnp.float32),
            pltpu.VMEM((_NBUF, _RB, W), jnp.float32),
            pltpu.SMEM((1,), jnp.float32),
            pltpu.SemaphoreType.DMA((_NBUF,)),
            pltpu.SemaphoreType.DMA((_NBUF,)),
        ],
    )(out2d, tgt2d)

    npk = peaks.shape[0]
    res = pl.pallas_call(
        _pos_body,
        in_specs=[
            pl.BlockSpec((npk // 128, 128), lambda: (0, 0)),
            pl.BlockSpec((npk // 128, 128), lambda: (0, 0)),
            pl.BlockSpec(memory_space=pltpu.SMEM),
        ],
        out_specs=pl.BlockSpec(memory_space=pltpu.SMEM),
        out_shape=jax.ShapeDtypeStruct((1,), jnp.float32),
    )(peaks.reshape(npk // 128, 128), maskp.reshape(npk // 128, 128), neg)
    return res[0]
